# Initial kernel scaffold; baseline (speedup 1.0000x reference)
import jax
import jax.numpy as jnp
from jax.experimental import pallas as pl


def kernel(img, i, j):
    raise NotImplementedError("write your pallas kernel here")


# speedup vs baseline: 8.4348x; 8.4348x over previous
import functools

import jax
import jax.numpy as jnp
from jax import lax
from jax.experimental import pallas as pl
from jax.experimental.pallas import tpu as pltpu
from jax.experimental.pallas import tpu_sc as plsc

B, C, H, W = 4, 96, 512, 512
TH, TW = 448, 448
NC, NS = 2, 16
NW = NC * NS
CH_PER_W = (B * C) // NW
CR = 64
NCHUNK = TH // CR
NSLOT = 3


def _crop_body(rows_hbm, meta_hbm, out_hbm, meta_v, buf0, buf1, buf2, rsems, wsems):
    bufs = (buf0, buf1, buf2)
    wid = lax.axis_index("s") * NC + lax.axis_index("c")
    pltpu.sync_copy(meta_hbm, meta_v)

    def channel(t, carry):
        bc = wid * CH_PER_W + t
        b = bc // C
        bvec = jnp.zeros((16,), jnp.int32) + b
        i_s = jnp.max(plsc.load_gather(meta_v, [bvec]))
        j_s = jnp.max(plsc.load_gather(meta_v, [bvec + 4]))
        j_al = pl.multiple_of(j_s & ~7, 8)
        j_off = j_s & 7
        row0 = bc * H + i_s
        orow0 = bc * TH

        def read(k):
            s = k % NSLOT
            return pltpu.async_copy(
                rows_hbm.at[pl.ds(row0 + k * CR, CR), pl.ds(j_al, TW + 8)],
                bufs[s],
                rsems.at[s],
            )

        def write(k):
            s = k % NSLOT
            return pltpu.async_copy(
                bufs[s].at[:, pl.ds(0, TW)],
                out_hbm.at[pl.ds(orow0 + k * CR, CR)],
                wsems.at[s],
            )

        rh = {0: read(0), 1: read(1), 2: read(2)}
        wh = {}
        for k in range(NCHUNK):
            if k >= 1:
                wh[k - 1].wait()
                if k + 2 < NCHUNK:
                    rh[k + 2] = read(k + 2)
            rh[k].wait()
            buf = bufs[k % NSLOT]

            @pl.when(j_off != 0)
            def _shift():
                def shift_row(r, c):
                    for tt in range(TW // 16):
                        v = buf[r, pl.ds(j_off + 16 * tt, 16)]
                        buf[r, pl.ds(16 * tt, 16)] = v
                    return c

                lax.fori_loop(0, CR, shift_row, 0)

            wh[k] = write(k)
        wh[NCHUNK - 1].wait()
        return carry

    lax.fori_loop(0, CH_PER_W, channel, 0)


def kernel(img, i, j):
    rows = img.reshape(B * C * H, W)
    meta = jnp.concatenate(
        [i.astype(jnp.int32), j.astype(jnp.int32), jnp.zeros((8,), jnp.int32)]
    )
    mesh = plsc.VectorSubcoreMesh(core_axis_name="c", subcore_axis_name="s")
    out = pl.kernel(
        _crop_body,
        mesh=mesh,
        out_type=jax.ShapeDtypeStruct((B * C * TH, TW), jnp.float32),
        scratch_types=[
            pltpu.VMEM((16,), jnp.int32),
            pltpu.VMEM((CR, TW + 8), jnp.float32),
            pltpu.VMEM((CR, TW + 8), jnp.float32),
            pltpu.VMEM((CR, TW + 8), jnp.float32),
            pltpu.SemaphoreType.DMA((NSLOT,)),
            pltpu.SemaphoreType.DMA((NSLOT,)),
        ],
        compiler_params=pltpu.CompilerParams(
            use_tc_tiling_on_sc=False, needs_layout_passes=False
        ),
    )(rows, meta)
    return out.reshape(B, C, TH, TW)


# speedup vs baseline: 13.2218x; 1.5675x over previous
import functools

import jax
import jax.numpy as jnp
from jax import lax
from jax.experimental import pallas as pl
from jax.experimental.pallas import tpu as pltpu
from jax.experimental.pallas import tpu_sc as plsc

B, C, H, W = 4, 96, 512, 512
TH, TW = 448, 448
NC, NS = 2, 16
NW = NC * NS
CH_PER_W = (B * C) // NW
CR = 112
NCHUNK = TH // CR
NSLOT = 2


def _crop_body(rows_hbm, meta_hbm, out_hbm, meta_v, buf0, buf1, rsems, wsems):
    bufs = (buf0, buf1)
    wid = lax.axis_index("s") * NC + lax.axis_index("c")
    pltpu.sync_copy(meta_hbm, meta_v)

    def channel(t, carry):
        bc = wid * CH_PER_W + t
        b = bc // C
        bvec = jnp.zeros((16,), jnp.int32) + b
        i_s = jnp.max(plsc.load_gather(meta_v, [bvec]))
        j_s = jnp.max(plsc.load_gather(meta_v, [bvec + 4]))
        j_al = pl.multiple_of(j_s & ~7, 8)
        j_off = j_s & 7
        row0 = bc * H + i_s
        orow0 = bc * TH

        def read(k):
            s = k % NSLOT
            return pltpu.async_copy(
                rows_hbm.at[pl.ds(row0 + k * CR, CR), pl.ds(j_al, TW + 8)],
                bufs[s],
                rsems.at[s],
            )

        def write(k):
            s = k % NSLOT
            return pltpu.async_copy(
                bufs[s].at[:, pl.ds(0, TW)],
                out_hbm.at[pl.ds(orow0 + k * CR, CR)],
                wsems.at[s],
            )

        rh = {k: read(k) for k in range(min(NSLOT, NCHUNK))}
        wh = {}
        for k in range(NCHUNK):
            if k >= 1:
                wh[k - 1].wait()
                if k - 1 + NSLOT < NCHUNK:
                    rh[k - 1 + NSLOT] = read(k - 1 + NSLOT)
            rh[k].wait()
            buf = bufs[k % NSLOT]

            @pl.when(j_off != 0)
            def _shift():
                @plsc.parallel_loop(0, CR, step=1, unroll=2)
                def shift_row(r):
                    for tt in range(TW // 16):
                        v = buf[r, pl.ds(j_off + 16 * tt, 16)]
                        buf[r, pl.ds(16 * tt, 16)] = v

            wh[k] = write(k)
        wh[NCHUNK - 1].wait()
        return carry

    lax.fori_loop(0, CH_PER_W, channel, 0)


def kernel(img, i, j):
    rows = img.reshape(B * C * H, W)
    meta = jnp.concatenate(
        [i.astype(jnp.int32), j.astype(jnp.int32), jnp.zeros((8,), jnp.int32)]
    )
    mesh = plsc.VectorSubcoreMesh(core_axis_name="c", subcore_axis_name="s")
    out = pl.kernel(
        _crop_body,
        mesh=mesh,
        out_type=jax.ShapeDtypeStruct((B * C * TH, TW), jnp.float32),
        scratch_types=[
            pltpu.VMEM((16,), jnp.int32),
            pltpu.VMEM((CR, TW + 8), jnp.float32),
            pltpu.VMEM((CR, TW + 8), jnp.float32),
            pltpu.SemaphoreType.DMA((NSLOT,)),
            pltpu.SemaphoreType.DMA((NSLOT,)),
        ],
        compiler_params=pltpu.CompilerParams(
            use_tc_tiling_on_sc=False, needs_layout_passes=False
        ),
    )(rows, meta)
    return out.reshape(B, C, TH, TW)


# speedup vs baseline: 13.5877x; 1.0277x over previous
import functools

import jax
import jax.numpy as jnp
from jax import lax
from jax.experimental import pallas as pl
from jax.experimental.pallas import tpu as pltpu
from jax.experimental.pallas import tpu_sc as plsc

B, C, H, W = 4, 96, 512, 512
TH, TW = 448, 448
NC, NS = 2, 16
NW = NC * NS
CH_PER_W = (B * C) // NW
CR = 64
NCHUNK = TH // CR
RW = TW + 8


def _crop_body(rows_hbm, meta_hbm, out_hbm, meta_v, in0, in1, ob0, ob1, rsems, wsems):
    ibufs = (in0, in1)
    obufs = (ob0, ob1)
    wid = lax.axis_index("s") * NC + lax.axis_index("c")
    pltpu.sync_copy(meta_hbm, meta_v)

    def channel(t, carry):
        bc = wid * CH_PER_W + t
        b = bc // C
        bvec = jnp.zeros((16,), jnp.int32) + b
        i_s = jnp.max(plsc.load_gather(meta_v, [bvec]))
        j_s = jnp.max(plsc.load_gather(meta_v, [bvec + 4]))
        j_al = pl.multiple_of(j_s & ~7, 8)
        j_off = j_s & 7
        row0 = bc * H + i_s
        orow0 = bc * TH

        def read(k):
            return pltpu.async_copy(
                rows_hbm.at[pl.ds(row0 + k * CR, CR), pl.ds(j_al, RW)],
                ibufs[k % 2],
                rsems.at[k % 2],
            )

        def write(k):
            return pltpu.async_copy(
                obufs[k % 2],
                out_hbm.at[pl.ds(orow0 + k * CR, CR)],
                wsems.at[k % 2],
            )

        rh = {0: read(0), 1: read(1)}
        wh = {}
        for k in range(NCHUNK):
            rh[k].wait()
            if k >= 2:
                wh[k - 2].wait()
            ibuf = ibufs[k % 2]
            obuf = obufs[k % 2]

            @plsc.parallel_loop(0, CR, step=1, unroll=2)
            def shift_row(r):
                for tt in range(TW // 16):
                    v = ibuf[r, pl.ds(j_off + 16 * tt, 16)]
                    obuf[r, pl.ds(16 * tt, 16)] = v

            if k + 2 < NCHUNK:
                rh[k + 2] = read(k + 2)
            wh[k] = write(k)
        wh[NCHUNK - 2].wait()
        wh[NCHUNK - 1].wait()
        return carry

    lax.fori_loop(0, CH_PER_W, channel, 0)


def kernel(img, i, j):
    rows = img.reshape(B * C * H, W)
    meta = jnp.concatenate(
        [i.astype(jnp.int32), j.astype(jnp.int32), jnp.zeros((8,), jnp.int32)]
    )
    mesh = plsc.VectorSubcoreMesh(core_axis_name="c", subcore_axis_name="s")
    out = pl.kernel(
        _crop_body,
        mesh=mesh,
        out_type=jax.ShapeDtypeStruct((B * C * TH, TW), jnp.float32),
        scratch_types=[
            pltpu.VMEM((16,), jnp.int32),
            pltpu.VMEM((CR, RW), jnp.float32),
            pltpu.VMEM((CR, RW), jnp.float32),
            pltpu.VMEM((CR, TW), jnp.float32),
            pltpu.VMEM((CR, TW), jnp.float32),
            pltpu.SemaphoreType.DMA((2,)),
            pltpu.SemaphoreType.DMA((2,)),
        ],
        compiler_params=pltpu.CompilerParams(
            use_tc_tiling_on_sc=False, needs_layout_passes=False
        ),
    )(rows, meta)
    return out.reshape(B, C, TH, TW)
